# baseline (speedup 1.0000x reference)
import functools

import jax
import jax.numpy as jnp
from jax import lax
from jax.experimental import pallas as pl
from jax.experimental.pallas import tpu as pltpu
from jax.experimental.pallas import tpu_sc as plsc

VOCAB = 100000
D = 128
B = 1024
L = 200
EPS = 1e-12

NC = 2
NS = 16
NW = NC * NS
NBUF = 2

SPLITS = (256, 256, 256, 256)
CH = 64
S = 64


@functools.cache
def _make_sc_gather(tok_call):
    tok_w = tok_call // NW
    n_chunks = tok_w // CH
    assert tok_w % CH == 0

    def body(ids_hbm, table_hbm, out_hbm, idx_v, rows_v, sem0, sem1):
        wid = lax.axis_index("s") * NC + lax.axis_index("c")
        out_base = wid * tok_w
        sems = (sem0, sem1)
        pltpu.sync_copy(ids_hbm.at[wid], idx_v)

        def start(chunk, buf):
            pltpu.async_copy(
                table_hbm.at[idx_v.at[chunk]], rows_v.at[buf], sems[buf]
            )

        def finish(chunk, buf):
            pltpu.make_async_copy(
                table_hbm.at[idx_v.at[chunk]], rows_v.at[buf], sems[buf]
            ).wait()
            pltpu.sync_copy(
                rows_v.at[buf], out_hbm.at[pl.ds(out_base + chunk * CH, CH)]
            )

        for bb in range(NBUF):
            start(bb, bb)

        def step(c, carry):
            for bb in range(NBUF):
                chunk = c + bb
                finish(chunk, bb)

                @pl.when(chunk + NBUF < n_chunks)
                def _():
                    start(chunk + NBUF, bb)

            return carry

        lax.fori_loop(0, n_chunks // NBUF, lambda i, cy: step(i * NBUF, cy),
                      0, unroll=False)
        for chunk in range(n_chunks - (n_chunks % NBUF), n_chunks):
            finish(chunk, chunk % NBUF)

    mesh = plsc.VectorSubcoreMesh(
        core_axis_name="c", subcore_axis_name="s", num_cores=NC, num_subcores=NS
    )
    return functools.partial(
        pl.kernel,
        out_type=jax.ShapeDtypeStruct((tok_call, D), jnp.float32),
        mesh=mesh,
        scratch_types=[
            pltpu.VMEM((n_chunks, CH), jnp.int32),
            pltpu.VMEM((NBUF, CH, D), jnp.float32),
            pltpu.SemaphoreType.DMA,
            pltpu.SemaphoreType.DMA,
        ],
    )(body)


def _tc_posln_compute(x_ref, w_ref, b_ref, o_ref, pos_scr):
    @pl.when(pl.program_id(0) == 0)
    def _():
        l_idx = lax.broadcasted_iota(jnp.int32, (L, D // 2), 0).astype(jnp.float32)
        j_idx = lax.broadcasted_iota(jnp.int32, (L, D // 2), 1).astype(jnp.float32)
        inv_freq = jnp.exp(j_idx * (-2.0 / D * jnp.log(10000.0)))
        angle = (L - 1.0 - l_idx) * inv_freq
        pos_scr[...] = jnp.concatenate([jnp.sin(angle), jnp.cos(angle)], axis=-1)

    x = x_ref[...]
    e = (x + pos_scr[...][None, :, :]).reshape(S * L, D)
    w_red = jnp.where(
        lax.broadcasted_iota(jnp.int32, (D, 8), 1) == 0, 1.0 / D, 0.0
    )
    u = lax.dot_general(
        e, w_red, (((1,), (0,)), ((), ())), preferred_element_type=jnp.float32
    )[:, 0:1]
    s2 = lax.dot_general(
        e * e, w_red, (((1,), (0,)), ((), ())),
        preferred_element_type=jnp.float32,
    )[:, 0:1]
    var = s2 - u * u
    y = (e - u) * lax.rsqrt(var + EPS)
    o_ref[...] = (
        y * w_ref[...][None, :] + b_ref[...][None, :]
    ).reshape(S, L, D)


def _tc_posln_first(x_ref, w_ref, b_ref, o_ref, pos_scr):
    _tc_posln_compute(x_ref, w_ref, b_ref, o_ref, pos_scr)


def _tc_posln_chained(prev_ref, x_ref, w_ref, b_ref, o_ref, pos_scr):
    del prev_ref
    _tc_posln_compute(x_ref, w_ref, b_ref, o_ref, pos_scr)


def _tc_posln_stage(x, prev, seq_base, nseq, ln_weight, ln_bias):
    nblk = nseq // S
    base = seq_base // S
    x_spec = pl.BlockSpec((S, L, D), lambda i: (i, 0, 0))
    wb_spec = pl.BlockSpec((D,), lambda i: (0,))
    out_spec = pl.BlockSpec((S, L, D), lambda i, _b=base: (i + _b, 0, 0))
    common = dict(
        grid=(nblk,),
        out_specs=out_spec,
        out_shape=jax.ShapeDtypeStruct((B, L, D), jnp.float32),
        scratch_shapes=[pltpu.VMEM((L, D), jnp.float32)],
    )
    if prev is None:
        return pl.pallas_call(
            _tc_posln_first,
            in_specs=[x_spec, wb_spec, wb_spec],
            **common,
        )(x, ln_weight, ln_bias)
    return pl.pallas_call(
        _tc_posln_chained,
        in_specs=[pl.BlockSpec(memory_space=pl.ANY), x_spec, wb_spec, wb_spec],
        input_output_aliases={0: 0},
        **common,
    )(prev, x, ln_weight, ln_bias)


def kernel(input_ids, table, ln_weight, ln_bias):
    flat_ids = input_ids.astype(jnp.int32).reshape(-1)
    out = None
    seq_base = 0
    for nseq in SPLITS:
        tok = nseq * L
        ids_c = lax.dynamic_slice(flat_ids, (seq_base * L,), (tok,)).reshape(
            NW, tok // (NW * CH), CH
        )
        rows = _make_sc_gather(tok)(ids_c, table)
        out = _tc_posln_stage(
            rows.reshape(nseq, L, D), out, seq_base, nseq, ln_weight, ln_bias
        )
        seq_base += nseq
    return out


# speedup vs baseline: 1.0708x; 1.0708x over previous
import functools

import jax
import jax.numpy as jnp
from jax import lax
from jax.experimental import pallas as pl
from jax.experimental.pallas import tpu as pltpu
from jax.experimental.pallas import tpu_sc as plsc

VOCAB = 100000
D = 128
B = 1024
L = 200
EPS = 1e-12

NC = 2
NS = 16
NW = NC * NS
NBUF = 4

SPLITS = (512, 512)
CH = 128
S = 64


@functools.cache
def _make_sc_gather(tok_call):
    tok_w = tok_call // NW
    n_chunks = tok_w // CH
    assert tok_w % CH == 0

    def body(ids_hbm, table_hbm, out_hbm, idx_v, rows_v, *sems):
        wid = lax.axis_index("s") * NC + lax.axis_index("c")
        out_base = wid * tok_w
        pltpu.sync_copy(ids_hbm.at[wid], idx_v)

        def start(chunk, buf):
            pltpu.async_copy(
                table_hbm.at[idx_v.at[chunk]], rows_v.at[buf], sems[buf]
            )

        def finish(chunk, buf):
            pltpu.make_async_copy(
                table_hbm.at[idx_v.at[chunk]], rows_v.at[buf], sems[buf]
            ).wait()
            pltpu.sync_copy(
                rows_v.at[buf], out_hbm.at[pl.ds(out_base + chunk * CH, CH)]
            )

        for bb in range(NBUF):
            start(bb, bb)

        def step(c, carry):
            for bb in range(NBUF):
                chunk = c + bb
                finish(chunk, bb)

                @pl.when(chunk + NBUF < n_chunks)
                def _():
                    start(chunk + NBUF, bb)

            return carry

        lax.fori_loop(0, n_chunks // NBUF, lambda i, cy: step(i * NBUF, cy),
                      0, unroll=False)
        for chunk in range(n_chunks - (n_chunks % NBUF), n_chunks):
            finish(chunk, chunk % NBUF)

    mesh = plsc.VectorSubcoreMesh(
        core_axis_name="c", subcore_axis_name="s", num_cores=NC, num_subcores=NS
    )
    return functools.partial(
        pl.kernel,
        out_type=jax.ShapeDtypeStruct((tok_call, D), jnp.float32),
        mesh=mesh,
        scratch_types=[
            pltpu.VMEM((n_chunks, CH), jnp.int32),
            pltpu.VMEM((NBUF, CH, D), jnp.float32),
        ] + [pltpu.SemaphoreType.DMA] * NBUF,
    )(body)


def _tc_posln_compute(x_ref, w_ref, b_ref, o_ref, pos_scr):
    @pl.when(pl.program_id(0) == 0)
    def _():
        l_idx = lax.broadcasted_iota(jnp.int32, (L, D // 2), 0).astype(jnp.float32)
        j_idx = lax.broadcasted_iota(jnp.int32, (L, D // 2), 1).astype(jnp.float32)
        inv_freq = jnp.exp(j_idx * (-2.0 / D * jnp.log(10000.0)))
        angle = (L - 1.0 - l_idx) * inv_freq
        pos_scr[...] = jnp.concatenate([jnp.sin(angle), jnp.cos(angle)], axis=-1)

    x = x_ref[...]
    e = (x + pos_scr[...][None, :, :]).reshape(S * L, D)
    w_red = jnp.where(
        lax.broadcasted_iota(jnp.int32, (D, 8), 1) == 0, 1.0 / D, 0.0
    )
    u = lax.dot_general(
        e, w_red, (((1,), (0,)), ((), ())), preferred_element_type=jnp.float32
    )[:, 0:1]
    s2 = lax.dot_general(
        e * e, w_red, (((1,), (0,)), ((), ())),
        preferred_element_type=jnp.float32,
    )[:, 0:1]
    var = s2 - u * u
    y = (e - u) * lax.rsqrt(var + EPS)
    o_ref[...] = (
        y * w_ref[...][None, :] + b_ref[...][None, :]
    ).reshape(S, L, D)


def _tc_posln_first(x_ref, w_ref, b_ref, o_ref, pos_scr):
    _tc_posln_compute(x_ref, w_ref, b_ref, o_ref, pos_scr)


def _tc_posln_chained(prev_ref, x_ref, w_ref, b_ref, o_ref, pos_scr):
    del prev_ref
    _tc_posln_compute(x_ref, w_ref, b_ref, o_ref, pos_scr)


def _tc_posln_stage(x, prev, seq_base, nseq, ln_weight, ln_bias):
    nblk = nseq // S
    base = seq_base // S
    x_spec = pl.BlockSpec((S, L, D), lambda i: (i, 0, 0))
    wb_spec = pl.BlockSpec((D,), lambda i: (0,))
    out_spec = pl.BlockSpec((S, L, D), lambda i, _b=base: (i + _b, 0, 0))
    common = dict(
        grid=(nblk,),
        out_specs=out_spec,
        out_shape=jax.ShapeDtypeStruct((B, L, D), jnp.float32),
        scratch_shapes=[pltpu.VMEM((L, D), jnp.float32)],
    )
    if prev is None:
        return pl.pallas_call(
            _tc_posln_first,
            in_specs=[x_spec, wb_spec, wb_spec],
            **common,
        )(x, ln_weight, ln_bias)
    return pl.pallas_call(
        _tc_posln_chained,
        in_specs=[pl.BlockSpec(memory_space=pl.ANY), x_spec, wb_spec, wb_spec],
        input_output_aliases={0: 0},
        **common,
    )(prev, x, ln_weight, ln_bias)


def kernel(input_ids, table, ln_weight, ln_bias):
    flat_ids = input_ids.astype(jnp.int32).reshape(-1)
    out = None
    seq_base = 0
    for nseq in SPLITS:
        tok = nseq * L
        ids_c = lax.dynamic_slice(flat_ids, (seq_base * L,), (tok,)).reshape(
            NW, tok // (NW * CH), CH
        )
        rows = _make_sc_gather(tok)(ids_c, table)
        out = _tc_posln_stage(
            rows.reshape(nseq, L, D), out, seq_base, nseq, ln_weight, ln_bias
        )
        seq_base += nseq
    return out
